# baseline (speedup 1.0000x reference)
import functools

import jax
import jax.numpy as jnp
from jax import lax
from jax.experimental import pallas as pl
from jax.experimental.pallas import tpu as pltpu
from jax.experimental.pallas import tpu_sc as plsc

_CH = 128


@functools.partial(jax.jit, static_argnums=(3,))
def _biased_embedding(index, vw_flat, bias_flat, D):
    (VD,) = vw_flat.shape
    V = VD // D
    (B,) = index.shape
    info = plsc.get_sparse_core_info()
    nw = info.num_cores * info.num_subcores
    b_per_w = B // nw
    nch = b_per_w // _CH

    mesh = plsc.VectorSubcoreMesh(core_axis_name="core", subcore_axis_name="subcore")

    scratch = (
        [pltpu.VMEM((_CH,), jnp.int32) for _ in range(nch)]
        + [pltpu.VMEM((D, _CH), jnp.float32) for _ in range(nch)]
        + [pltpu.VMEM((_CH,), jnp.float32) for _ in range(nch)]
        + [pltpu.SemaphoreType.DMA]
    )

    @functools.partial(
        pl.kernel,
        out_type=(
            jax.ShapeDtypeStruct((B,), jnp.float32),
            jax.ShapeDtypeStruct((D, B), jnp.float32),
        ),
        mesh=mesh,
        scratch_types=scratch,
        compiler_params=pltpu.CompilerParams(use_tc_tiling_on_sc=False),
    )
    def run(vw_hbm, bias_hbm, idx_hbm, bias_out, vect_out, *bufs):
        idxb = bufs[:nch]
        vbuf = bufs[nch:2 * nch]
        bbuf = bufs[2 * nch:3 * nch]
        sem = bufs[3 * nch]

        wid = lax.axis_index("subcore") * info.num_cores + lax.axis_index("core")
        base = wid * b_per_w

        copies = []
        for w in range(nch):
            wb = base + w * _CH
            pltpu.sync_copy(idx_hbm.at[pl.ds(wb, _CH)], idxb[w])
            for d in range(D):
                sub = vw_hbm.at[pl.ds(d * V, V)]
                copies.append(
                    pltpu.async_copy(sub.at[idxb[w]], vbuf[w].at[d], sem)
                )
            copies.append(pltpu.async_copy(bias_hbm.at[idxb[w]], bbuf[w], sem))
        for c in copies:
            c.wait()
        for w in range(nch):
            wb = base + w * _CH
            pltpu.sync_copy(vbuf[w], vect_out.at[:, pl.ds(wb, _CH)])
            pltpu.sync_copy(bbuf[w], bias_out.at[pl.ds(wb, _CH)])

    bias, vect_t = run(vw_flat, bias_flat, index)
    return bias, vect_t.T


def kernel(index, vect_weight, bias_weight):
    return _biased_embedding(
        index.astype(jnp.int32),
        vect_weight.T.reshape(-1),
        bias_weight.reshape(-1),
        vect_weight.shape[1],
    )


# speedup vs baseline: 4.9933x; 4.9933x over previous
import functools

import jax
import jax.numpy as jnp
from jax import lax
from jax.experimental import pallas as pl
from jax.experimental.pallas import tpu as pltpu
from jax.experimental.pallas import tpu_sc as plsc

_CH = 128


@jax.jit
def _biased_embedding(index, vect_weight, bias_flat):
    V, D = vect_weight.shape
    (B,) = index.shape
    info = plsc.get_sparse_core_info()
    nw = info.num_cores * info.num_subcores
    b_per_w = B // nw
    nch = b_per_w // _CH

    mesh = plsc.VectorSubcoreMesh(core_axis_name="core", subcore_axis_name="subcore")

    scratch = (
        [pltpu.VMEM((_CH,), jnp.int32) for _ in range(nch)]
        + [pltpu.VMEM((_CH, D), jnp.float32) for _ in range(nch)]
        + [pltpu.VMEM((_CH,), jnp.float32) for _ in range(nch)]
        + [pltpu.SemaphoreType.DMA]
    )

    @functools.partial(
        pl.kernel,
        out_type=(
            jax.ShapeDtypeStruct((B,), jnp.float32),
            jax.ShapeDtypeStruct((B, D), jnp.float32),
        ),
        mesh=mesh,
        scratch_types=scratch,
        compiler_params=pltpu.CompilerParams(use_tc_tiling_on_sc=False),
    )
    def run(vect_hbm, bias_hbm, idx_hbm, bias_out, vect_out, *bufs):
        idxb = bufs[:nch]
        vbuf = bufs[nch:2 * nch]
        bbuf = bufs[2 * nch:3 * nch]
        sem = bufs[3 * nch]

        wid = lax.axis_index("subcore") * info.num_cores + lax.axis_index("core")
        base = wid * b_per_w

        copies = []
        for w in range(nch):
            wb = base + w * _CH
            pltpu.sync_copy(idx_hbm.at[pl.ds(wb, _CH)], idxb[w])
            copies.append(pltpu.async_copy(vect_hbm.at[idxb[w]], vbuf[w], sem))
            copies.append(pltpu.async_copy(bias_hbm.at[idxb[w]], bbuf[w], sem))
        for c in copies:
            c.wait()
        for w in range(nch):
            wb = base + w * _CH
            pltpu.sync_copy(vbuf[w], vect_out.at[pl.ds(wb, _CH)])
            pltpu.sync_copy(bbuf[w], bias_out.at[pl.ds(wb, _CH)])

    return run(vect_weight, bias_flat, index)


def kernel(index, vect_weight, bias_weight):
    return _biased_embedding(
        index.astype(jnp.int32), vect_weight, bias_weight.reshape(-1)
    )
